# baseline (speedup 1.0000x reference)
import functools

import jax
import jax.numpy as jnp
from jax import lax
from jax.experimental import pallas as pl
from jax.experimental.pallas import tpu as pltpu
from jax.experimental.pallas import tpu_sc as plsc

N = 10000
E = 160000
NG = 64
NUM_RADIAL = 128
EMB_RBF = 16
D_ATOM = 256
D_EDGE = 512
N_BLOCKS = 4
CUTOFF = 12.0

TA = 256
NT = 40
NPAD = TA * NT
EP = 163840
TE = 1024
CE = 512
NW = 32
INV_SQRT2 = 0.7071067811865475


def _sc_gather(table, idx, chunk):
    row_shape = table.shape[1:]
    B = idx.shape[0]
    per_w = B // NW
    n_iter = per_w // chunk
    assert per_w % chunk == 0 and B % NW == 0 and per_w % 8 == 0

    def body(table_hbm, idx_hbm, out_hbm, idx_v, rows_v, sem):
        wid = lax.axis_index("s") * 2 + lax.axis_index("c")
        base = wid * per_w

        def step(j, carry):
            off = base + j * chunk
            pltpu.sync_copy(idx_hbm.at[pl.ds(off, chunk)], idx_v)
            pltpu.async_copy(table_hbm.at[idx_v], rows_v, sem).wait()
            pltpu.sync_copy(rows_v, out_hbm.at[pl.ds(off, chunk)])
            return carry

        lax.fori_loop(0, n_iter, step, 0)

    mesh = plsc.VectorSubcoreMesh(core_axis_name="c", subcore_axis_name="s")
    fn = pl.kernel(
        body,
        out_type=jax.ShapeDtypeStruct((B,) + row_shape, table.dtype),
        mesh=mesh,
        scratch_types=[
            pltpu.VMEM((chunk,), jnp.int32),
            pltpu.VMEM((chunk,) + row_shape, table.dtype),
            pltpu.SemaphoreType.DMA,
        ],
    )
    return fn(table, idx)


def _edge_init_body(hs_ref, hd_ref, wr_ref, ws_ref, wd_ref,
                    we_ref, m_ref, re_ref):
    vec = hd_ref[:, D_ATOM:] - hs_ref[:, D_ATOM:]
    d2 = jnp.sum(vec * vec, axis=1, keepdims=True)
    dist = jnp.sqrt(d2 + 1e-12) + 1e-6
    n = lax.broadcasted_iota(jnp.int32, (1, NUM_RADIAL), 1).astype(jnp.float32) + 1.0
    rbf = jnp.sqrt(2.0 / CUTOFF) * jnp.sin(n * (jnp.pi / CUTOFF) * dist) / dist
    u = jnp.clip(dist / CUTOFF, 0.0, 1.0)
    u5 = u * u * u * u * u
    env = 1.0 + (-21.0) * u5 + 35.0 * (u5 * u) + (-15.0) * (u5 * u * u)
    env = jnp.where(u < 1.0, env, 0.0)
    rbf = rbf * env
    re = jnp.dot(rbf.astype(jnp.bfloat16), wr_ref[...],
                 preferred_element_type=jnp.float32)
    z = (jnp.dot(hs_ref[:, :D_ATOM].astype(jnp.bfloat16), ws_ref[...],
                 preferred_element_type=jnp.float32)
         + jnp.dot(hd_ref[:, :D_ATOM].astype(jnp.bfloat16), wd_ref[...],
                   preferred_element_type=jnp.float32)
         + jnp.dot(re.astype(jnp.bfloat16), we_ref[...],
                   preferred_element_type=jnp.float32))
    m_ref[...] = (z * jax.nn.sigmoid(z)).astype(jnp.bfloat16)
    re_ref[...] = re


def _edge_init(hs, hd, W_rbf, We_s, We_d, We_r):
    grid = (EP // TE,)
    return pl.pallas_call(
        _edge_init_body,
        grid=grid,
        in_specs=[
            pl.BlockSpec((TE, D_ATOM + 128), lambda i: (i, 0)),
            pl.BlockSpec((TE, D_ATOM + 128), lambda i: (i, 0)),
            pl.BlockSpec((NUM_RADIAL, EMB_RBF), lambda i: (0, 0)),
            pl.BlockSpec((D_ATOM, D_EDGE), lambda i: (0, 0)),
            pl.BlockSpec((D_ATOM, D_EDGE), lambda i: (0, 0)),
            pl.BlockSpec((EMB_RBF, D_EDGE), lambda i: (0, 0)),
        ],
        out_specs=[
            pl.BlockSpec((TE, D_EDGE), lambda i: (i, 0)),
            pl.BlockSpec((TE, EMB_RBF), lambda i: (i, 0)),
        ],
        out_shape=[
            jax.ShapeDtypeStruct((EP, D_EDGE), jnp.bfloat16),
            jax.ShapeDtypeStruct((EP, EMB_RBF), jnp.float32),
        ],
    )(hs, hd, W_rbf, We_s, We_d, We_r)


def _edge_mix_body(m_ref, re_ref, wg_ref, w1_ref, w2_ref, wa_ref,
                   mmid_ref, amsg_ref):
    m0 = m_ref[...]
    gate = jnp.dot(re_ref[...].astype(jnp.bfloat16), wg_ref[...],
                   preferred_element_type=jnp.float32)
    z1 = jnp.dot(m0, w1_ref[...], preferred_element_type=jnp.float32)
    m2 = z1 * jax.nn.sigmoid(z1) * gate
    z2 = jnp.dot(m2.astype(jnp.bfloat16), w2_ref[...],
                 preferred_element_type=jnp.float32)
    m2 = z2 * jax.nn.sigmoid(z2)
    mm = (m0.astype(jnp.float32) + m2) * INV_SQRT2
    mmb = mm.astype(jnp.bfloat16)
    mmid_ref[...] = mmb
    amsg_ref[...] = jnp.dot(mmb, wa_ref[...],
                            preferred_element_type=jnp.float32).astype(jnp.bfloat16)


def _edge_mix(m, rbf_emb, Wgate, Wm1, Wm2, Wam):
    grid = (EP // TE,)
    return pl.pallas_call(
        _edge_mix_body,
        grid=grid,
        in_specs=[
            pl.BlockSpec((TE, D_EDGE), lambda i: (i, 0)),
            pl.BlockSpec((TE, EMB_RBF), lambda i: (i, 0)),
            pl.BlockSpec((EMB_RBF, D_EDGE), lambda i: (0, 0)),
            pl.BlockSpec((D_EDGE, D_EDGE), lambda i: (0, 0)),
            pl.BlockSpec((D_EDGE, D_EDGE), lambda i: (0, 0)),
            pl.BlockSpec((D_EDGE, D_ATOM), lambda i: (0, 0)),
        ],
        out_specs=[
            pl.BlockSpec((TE, D_EDGE), lambda i: (i, 0)),
            pl.BlockSpec((TE, D_ATOM), lambda i: (i, 0)),
        ],
        out_shape=[
            jax.ShapeDtypeStruct((EP, D_EDGE), jnp.bfloat16),
            jax.ShapeDtypeStruct((EP, D_ATOM), jnp.bfloat16),
        ],
    )(m, rbf_emb, Wgate, Wm1, Wm2, Wam)


def _atom_body(starts_ref, amsg_hbm, dst_hbm, h_ref, wh_ref, out_ref,
               amsg_v, dst_v, agg_ref, sem1, sem2):
    t = pl.program_id(0)
    start = starts_ref[t]
    end = starts_ref[t + 1]
    c0 = start // CE
    nch = jnp.maximum(0, (end + CE - 1) // CE - c0)
    agg_ref[...] = jnp.zeros((TA, D_ATOM), jnp.float32)

    def step(j, carry):
        off = pl.multiple_of((c0 + j) * CE, CE)
        c1 = pltpu.make_async_copy(amsg_hbm.at[pl.ds(off, CE)], amsg_v, sem1)
        c2 = pltpu.make_async_copy(dst_hbm.at[pl.ds(off, CE)], dst_v, sem2)
        c1.start()
        c2.start()
        c1.wait()
        c2.wait()
        dstl = dst_v[...] - t * TA
        ids = lax.broadcasted_iota(jnp.int32, (TA, CE), 0)
        S = (ids == dstl[None, :]).astype(jnp.bfloat16)
        agg_ref[...] += jnp.dot(S, amsg_v[...], preferred_element_type=jnp.float32)
        return carry

    lax.fori_loop(0, nch, step, 0)
    z = jnp.dot(agg_ref[...], wh_ref[...], preferred_element_type=jnp.float32)
    out_ref[...] = h_ref[...] + z * jax.nn.sigmoid(z)


def _atom_update(starts, amsg, dst_m, h, Wh):
    grid_spec = pltpu.PrefetchScalarGridSpec(
        num_scalar_prefetch=1,
        grid=(NT,),
        in_specs=[
            pl.BlockSpec(memory_space=pl.ANY),
            pl.BlockSpec(memory_space=pl.ANY),
            pl.BlockSpec((TA, D_ATOM), lambda t, starts: (t, 0)),
            pl.BlockSpec((D_ATOM, D_ATOM), lambda t, starts: (0, 0)),
        ],
        out_specs=pl.BlockSpec((TA, D_ATOM), lambda t, starts: (t, 0)),
        scratch_shapes=[
            pltpu.VMEM((CE, D_ATOM), jnp.bfloat16),
            pltpu.VMEM((CE,), jnp.int32),
            pltpu.VMEM((TA, D_ATOM), jnp.float32),
            pltpu.SemaphoreType.DMA,
            pltpu.SemaphoreType.DMA,
        ],
    )
    return pl.pallas_call(
        _atom_body,
        grid_spec=grid_spec,
        out_shape=jax.ShapeDtypeStruct((NPAD, D_ATOM), jnp.float32),
    )(starts, amsg, dst_m, h, Wh)


def _edge_up_body(hs_ref, hd_ref, m_ref, ws_ref, wd_ref, wm_ref, out_ref):
    z = (jnp.dot(hs_ref[...], ws_ref[...], preferred_element_type=jnp.float32)
         + jnp.dot(hd_ref[...], wd_ref[...], preferred_element_type=jnp.float32)
         + jnp.dot(m_ref[...], wm_ref[...], preferred_element_type=jnp.float32))
    e = z * jax.nn.sigmoid(z)
    out_ref[...] = ((m_ref[...].astype(jnp.float32) + e)
                    * INV_SQRT2).astype(jnp.bfloat16)


def _edge_up(hs, hd, mmid, We_s, We_d, We_m):
    grid = (EP // TE,)
    return pl.pallas_call(
        _edge_up_body,
        grid=grid,
        in_specs=[
            pl.BlockSpec((TE, D_ATOM), lambda i: (i, 0)),
            pl.BlockSpec((TE, D_ATOM), lambda i: (i, 0)),
            pl.BlockSpec((TE, D_EDGE), lambda i: (i, 0)),
            pl.BlockSpec((D_ATOM, D_EDGE), lambda i: (0, 0)),
            pl.BlockSpec((D_ATOM, D_EDGE), lambda i: (0, 0)),
            pl.BlockSpec((D_EDGE, D_EDGE), lambda i: (0, 0)),
        ],
        out_specs=pl.BlockSpec((TE, D_EDGE), lambda i: (i, 0)),
        out_shape=jax.ShapeDtypeStruct((EP, D_EDGE), jnp.bfloat16),
    )(hs, hd, mmid, We_s, We_d, We_m)


def _energy_body(h_ref, b_ref, w_ref, out_ref):
    t = pl.program_id(0)

    @pl.when(t == 0)
    def _():
        out_ref[...] = jnp.zeros_like(out_ref)

    e = jnp.dot(h_ref[...], w_ref[...], preferred_element_type=jnp.float32)
    bids = b_ref[0, 0, :]
    gids = lax.broadcasted_iota(jnp.int32, (NG, TA), 0)
    S = (gids == bids[None, :]).astype(jnp.float32)
    out_ref[...] += jnp.dot(S, e, preferred_element_type=jnp.float32)


def _energy(h, bids3, Wout_pad):
    return pl.pallas_call(
        _energy_body,
        grid=(NT,),
        in_specs=[
            pl.BlockSpec((TA, D_ATOM), lambda t: (t, 0)),
            pl.BlockSpec((1, 1, TA), lambda t: (t, 0, 0)),
            pl.BlockSpec((D_ATOM, 128), lambda t: (0, 0)),
        ],
        out_specs=pl.BlockSpec((NG, 128), lambda t: (0, 0)),
        out_shape=jax.ShapeDtypeStruct((NG, 128), jnp.float32),
    )(h, bids3, Wout_pad)


def kernel(atomic_numbers, pos, edge_index, batch_ids, emb_table, W_rbf,
           W_edge, W_m1, W_m2, W_gate, W_am, W_h, W_e, W_out):
    src = edge_index[0].astype(jnp.int32)
    dst = edge_index[1].astype(jnp.int32)
    perm = jnp.argsort(dst)
    dsts = dst[perm]
    srcs = src[perm]
    pad_e = EP - E
    zpad = jnp.zeros((pad_e,), jnp.int32)
    src_g = jnp.concatenate([srcs, zpad])
    dst_g = jnp.concatenate([dsts, zpad])
    dst_m = jnp.concatenate([dsts, jnp.full((pad_e,), 1 << 20, jnp.int32)])
    starts = jnp.searchsorted(
        dsts, jnp.arange(NT + 1, dtype=jnp.int32) * TA).astype(jnp.int32)

    an_pad = jnp.concatenate(
        [atomic_numbers.astype(jnp.int32), jnp.zeros((NPAD - N,), jnp.int32)])
    pos_pad = jnp.zeros((NPAD, 128), jnp.float32).at[:N, :3].set(pos)

    bf = jnp.bfloat16
    h = _sc_gather(emb_table, an_pad, 320)
    hp = jnp.concatenate([h, pos_pad], axis=1)
    hps = _sc_gather(hp, src_g, 256)
    hpd = _sc_gather(hp, dst_g, 256)

    m, rbf_emb = _edge_init(hps, hpd, W_rbf.astype(bf),
                            W_edge[:D_ATOM].astype(bf),
                            W_edge[D_ATOM:2 * D_ATOM].astype(bf),
                            W_edge[2 * D_ATOM:].astype(bf))

    for i in range(N_BLOCKS):
        mmid, amsg = _edge_mix(m, rbf_emb, W_gate[i].astype(bf),
                               W_m1[i].astype(bf), W_m2[i].astype(bf),
                               W_am[i].astype(bf))
        h = _atom_update(starts, amsg, dst_m, h, W_h[i])
        hpk = lax.bitcast_convert_type(
            h.astype(bf).reshape(NPAD, 128, 2), jnp.float32)
        hs = lax.bitcast_convert_type(
            _sc_gather(hpk, src_g, 256), bf).reshape(EP, D_ATOM)
        hd = lax.bitcast_convert_type(
            _sc_gather(hpk, dst_g, 256), bf).reshape(EP, D_ATOM)
        m = _edge_up(hs, hd, mmid, W_e[i][:D_ATOM].astype(bf),
                     W_e[i][D_ATOM:2 * D_ATOM].astype(bf),
                     W_e[i][2 * D_ATOM:].astype(bf))

    bids3 = jnp.concatenate(
        [batch_ids.astype(jnp.int32),
         jnp.full((NPAD - N,), NG, jnp.int32)]).reshape(NT, 1, TA)
    wout_pad = jnp.zeros((D_ATOM, 128), jnp.float32).at[:, :1].set(W_out)
    energy = _energy(h, bids3, wout_pad)
    return energy[:, 0]


# speedup vs baseline: 1.2100x; 1.2100x over previous
import functools

import jax
import jax.numpy as jnp
from jax import lax
from jax.experimental import pallas as pl
from jax.experimental.pallas import tpu as pltpu
from jax.experimental.pallas import tpu_sc as plsc

N = 10000
E = 160000
NG = 64
NUM_RADIAL = 128
EMB_RBF = 16
D_ATOM = 256
D_EDGE = 512
N_BLOCKS = 4
CUTOFF = 12.0

TA = 256
NT = 40
NPAD = TA * NT
EP = 163840
TE = 1024
CE = 512
NW = 32
INV_SQRT2 = 0.7071067811865475


def _sc_gather(table, idx, chunk):
    row_shape = table.shape[1:]
    B = idx.shape[0]
    per_w = B // NW
    n_iter = per_w // chunk
    assert per_w % chunk == 0 and B % NW == 0 and per_w % 8 == 0

    def body(table_hbm, idx_hbm, out_hbm, idx_v, rows_v, sem):
        wid = lax.axis_index("s") * 2 + lax.axis_index("c")
        base = wid * per_w

        def step(j, carry):
            off = base + j * chunk
            pltpu.sync_copy(idx_hbm.at[pl.ds(off, chunk)], idx_v)
            pltpu.async_copy(table_hbm.at[idx_v], rows_v, sem).wait()
            pltpu.sync_copy(rows_v, out_hbm.at[pl.ds(off, chunk)])
            return carry

        lax.fori_loop(0, n_iter, step, 0)

    mesh = plsc.VectorSubcoreMesh(core_axis_name="c", subcore_axis_name="s")
    fn = pl.kernel(
        body,
        out_type=jax.ShapeDtypeStruct((B,) + row_shape, table.dtype),
        mesh=mesh,
        scratch_types=[
            pltpu.VMEM((chunk,), jnp.int32),
            pltpu.VMEM((chunk,) + row_shape, table.dtype),
            pltpu.SemaphoreType.DMA,
        ],
    )
    return fn(table, idx)


def _edge_init_body(hs_ref, hd_ref, wr_ref, ws_ref, wd_ref,
                    we_ref, m_ref, re_ref):
    vec = hd_ref[:, D_ATOM:] - hs_ref[:, D_ATOM:]
    d2 = jnp.sum(vec * vec, axis=1, keepdims=True)
    dist = jnp.sqrt(d2 + 1e-12) + 1e-6
    n = lax.broadcasted_iota(jnp.int32, (1, NUM_RADIAL), 1).astype(jnp.float32) + 1.0
    rbf = jnp.sqrt(2.0 / CUTOFF) * jnp.sin(n * (jnp.pi / CUTOFF) * dist) / dist
    u = jnp.clip(dist / CUTOFF, 0.0, 1.0)
    u5 = u * u * u * u * u
    env = 1.0 + (-21.0) * u5 + 35.0 * (u5 * u) + (-15.0) * (u5 * u * u)
    env = jnp.where(u < 1.0, env, 0.0)
    rbf = rbf * env
    re = jnp.dot(rbf.astype(jnp.bfloat16), wr_ref[...],
                 preferred_element_type=jnp.float32)
    z = (jnp.dot(hs_ref[:, :D_ATOM].astype(jnp.bfloat16), ws_ref[...],
                 preferred_element_type=jnp.float32)
         + jnp.dot(hd_ref[:, :D_ATOM].astype(jnp.bfloat16), wd_ref[...],
                   preferred_element_type=jnp.float32)
         + jnp.dot(re.astype(jnp.bfloat16), we_ref[...],
                   preferred_element_type=jnp.float32))
    m_ref[...] = (z * jax.nn.sigmoid(z)).astype(jnp.bfloat16)
    re_ref[...] = re


def _edge_init(hs, hd, W_rbf, We_s, We_d, We_r):
    grid = (EP // TE,)
    return pl.pallas_call(
        _edge_init_body,
        grid=grid,
        in_specs=[
            pl.BlockSpec((TE, D_ATOM + 128), lambda i: (i, 0)),
            pl.BlockSpec((TE, D_ATOM + 128), lambda i: (i, 0)),
            pl.BlockSpec((NUM_RADIAL, EMB_RBF), lambda i: (0, 0)),
            pl.BlockSpec((D_ATOM, D_EDGE), lambda i: (0, 0)),
            pl.BlockSpec((D_ATOM, D_EDGE), lambda i: (0, 0)),
            pl.BlockSpec((EMB_RBF, D_EDGE), lambda i: (0, 0)),
        ],
        out_specs=[
            pl.BlockSpec((TE, D_EDGE), lambda i: (i, 0)),
            pl.BlockSpec((TE, EMB_RBF), lambda i: (i, 0)),
        ],
        out_shape=[
            jax.ShapeDtypeStruct((EP, D_EDGE), jnp.bfloat16),
            jax.ShapeDtypeStruct((EP, EMB_RBF), jnp.float32),
        ],
    )(hs, hd, W_rbf, We_s, We_d, We_r)


def _edge_mix_body(m_ref, re_ref, wg_ref, w1_ref, w2_ref, wa_ref,
                   mmid_ref, amsg_ref):
    m0 = m_ref[...]
    gate = jnp.dot(re_ref[...].astype(jnp.bfloat16), wg_ref[...],
                   preferred_element_type=jnp.float32)
    z1 = jnp.dot(m0, w1_ref[...], preferred_element_type=jnp.float32)
    m2 = z1 * jax.nn.sigmoid(z1) * gate
    z2 = jnp.dot(m2.astype(jnp.bfloat16), w2_ref[...],
                 preferred_element_type=jnp.float32)
    m2 = z2 * jax.nn.sigmoid(z2)
    mm = (m0.astype(jnp.float32) + m2) * INV_SQRT2
    mmb = mm.astype(jnp.bfloat16)
    mmid_ref[...] = mmb
    amsg_ref[...] = jnp.dot(mmb, wa_ref[...],
                            preferred_element_type=jnp.float32).astype(jnp.bfloat16)


def _edge_mix(m, rbf_emb, Wgate, Wm1, Wm2, Wam):
    grid = (EP // TE,)
    return pl.pallas_call(
        _edge_mix_body,
        grid=grid,
        in_specs=[
            pl.BlockSpec((TE, D_EDGE), lambda i: (i, 0)),
            pl.BlockSpec((TE, EMB_RBF), lambda i: (i, 0)),
            pl.BlockSpec((EMB_RBF, D_EDGE), lambda i: (0, 0)),
            pl.BlockSpec((D_EDGE, D_EDGE), lambda i: (0, 0)),
            pl.BlockSpec((D_EDGE, D_EDGE), lambda i: (0, 0)),
            pl.BlockSpec((D_EDGE, D_ATOM), lambda i: (0, 0)),
        ],
        out_specs=[
            pl.BlockSpec((TE, D_EDGE), lambda i: (i, 0)),
            pl.BlockSpec((TE, D_ATOM), lambda i: (i, 0)),
        ],
        out_shape=[
            jax.ShapeDtypeStruct((EP, D_EDGE), jnp.bfloat16),
            jax.ShapeDtypeStruct((EP, D_ATOM), jnp.bfloat16),
        ],
    )(m, rbf_emb, Wgate, Wm1, Wm2, Wam)


def _atom_body(starts_ref, amsg_hbm, dst_hbm, h_ref, wh_ref, hnew_ref,
               hdst_hbm, amsg_v, dst_v, agg_ref, hdc_ref, sem1, sem2, sem3):
    t = pl.program_id(0)
    start = starts_ref[t]
    end = starts_ref[t + 1]
    end = jnp.where(t == NT - 1, EP, end)
    c0 = start // CE
    nch = jnp.maximum(0, (end + CE - 1) // CE - c0)
    agg_ref[...] = jnp.zeros((TA, D_ATOM), jnp.float32)

    def step(j, carry):
        off = pl.multiple_of((c0 + j) * CE, CE)
        c1 = pltpu.make_async_copy(amsg_hbm.at[pl.ds(off, CE)], amsg_v, sem1)
        c2 = pltpu.make_async_copy(dst_hbm.at[pl.ds(off, CE)], dst_v, sem2)
        c1.start()
        c2.start()
        c1.wait()
        c2.wait()
        dstl = dst_v[...] - t * TA
        ids = lax.broadcasted_iota(jnp.int32, (TA, CE), 0)
        S = (ids == dstl[None, :]).astype(jnp.bfloat16)
        agg_ref[...] += jnp.dot(S, amsg_v[...], preferred_element_type=jnp.float32)
        return carry

    lax.fori_loop(0, nch, step, 0)
    z = jnp.dot(agg_ref[...], wh_ref[...], preferred_element_type=jnp.float32)
    hnew = h_ref[...] + z * jax.nn.sigmoid(z)
    hnew_ref[...] = hnew
    hb = hnew.astype(jnp.bfloat16)

    def step2(j, carry):
        off = pl.multiple_of((c0 + j) * CE, CE)
        c2 = pltpu.make_async_copy(dst_hbm.at[pl.ds(off, CE)], dst_v, sem2)
        c2.start()
        c2.wait()
        dstl = dst_v[...] - t * TA
        ids = lax.broadcasted_iota(jnp.int32, (CE, TA), 1)
        G = (ids == dstl[:, None]).astype(jnp.bfloat16)
        hd = jnp.dot(G, hb, preferred_element_type=jnp.float32).astype(jnp.bfloat16)
        shared = jnp.logical_and(j == 0, start % CE != 0)

        @pl.when(shared)
        def _():
            cr = pltpu.make_async_copy(hdst_hbm.at[pl.ds(off, CE)], hdc_ref, sem3)
            cr.start()
            cr.wait()
            hdc_ref[...] = hdc_ref[...] + hd

        @pl.when(jnp.logical_not(shared))
        def _():
            hdc_ref[...] = hd

        cw = pltpu.make_async_copy(hdc_ref, hdst_hbm.at[pl.ds(off, CE)], sem3)
        cw.start()
        cw.wait()
        return carry

    lax.fori_loop(0, nch, step2, 0)


def _atom_update(starts, amsg, dst_m, h, Wh):
    grid_spec = pltpu.PrefetchScalarGridSpec(
        num_scalar_prefetch=1,
        grid=(NT,),
        in_specs=[
            pl.BlockSpec(memory_space=pl.ANY),
            pl.BlockSpec(memory_space=pl.ANY),
            pl.BlockSpec((TA, D_ATOM), lambda t, starts: (t, 0)),
            pl.BlockSpec((D_ATOM, D_ATOM), lambda t, starts: (0, 0)),
        ],
        out_specs=[
            pl.BlockSpec((TA, D_ATOM), lambda t, starts: (t, 0)),
            pl.BlockSpec(memory_space=pl.ANY),
        ],
        scratch_shapes=[
            pltpu.VMEM((CE, D_ATOM), jnp.bfloat16),
            pltpu.VMEM((CE,), jnp.int32),
            pltpu.VMEM((TA, D_ATOM), jnp.float32),
            pltpu.VMEM((CE, D_ATOM), jnp.bfloat16),
            pltpu.SemaphoreType.DMA,
            pltpu.SemaphoreType.DMA,
            pltpu.SemaphoreType.DMA,
        ],
    )
    return pl.pallas_call(
        _atom_body,
        grid_spec=grid_spec,
        out_shape=[
            jax.ShapeDtypeStruct((NPAD, D_ATOM), jnp.float32),
            jax.ShapeDtypeStruct((EP, D_ATOM), jnp.bfloat16),
        ],
    )(starts, amsg, dst_m, h, Wh)


def _edge_up_body(hs_ref, hd_ref, m_ref, ws_ref, wd_ref, wm_ref, out_ref):
    z = (jnp.dot(hs_ref[...].astype(jnp.bfloat16), ws_ref[...],
                 preferred_element_type=jnp.float32)
         + jnp.dot(hd_ref[...], wd_ref[...], preferred_element_type=jnp.float32)
         + jnp.dot(m_ref[...], wm_ref[...], preferred_element_type=jnp.float32))
    e = z * jax.nn.sigmoid(z)
    out_ref[...] = ((m_ref[...].astype(jnp.float32) + e)
                    * INV_SQRT2).astype(jnp.bfloat16)


def _edge_up(hs, hd, mmid, We_s, We_d, We_m):
    grid = (EP // TE,)
    return pl.pallas_call(
        _edge_up_body,
        grid=grid,
        in_specs=[
            pl.BlockSpec((TE, D_ATOM), lambda i: (i, 0)),
            pl.BlockSpec((TE, D_ATOM), lambda i: (i, 0)),
            pl.BlockSpec((TE, D_EDGE), lambda i: (i, 0)),
            pl.BlockSpec((D_ATOM, D_EDGE), lambda i: (0, 0)),
            pl.BlockSpec((D_ATOM, D_EDGE), lambda i: (0, 0)),
            pl.BlockSpec((D_EDGE, D_EDGE), lambda i: (0, 0)),
        ],
        out_specs=pl.BlockSpec((TE, D_EDGE), lambda i: (i, 0)),
        out_shape=jax.ShapeDtypeStruct((EP, D_EDGE), jnp.bfloat16),
    )(hs, hd, mmid, We_s, We_d, We_m)


def _energy_body(h_ref, b_ref, w_ref, out_ref):
    t = pl.program_id(0)

    @pl.when(t == 0)
    def _():
        out_ref[...] = jnp.zeros_like(out_ref)

    e = jnp.dot(h_ref[...], w_ref[...], preferred_element_type=jnp.float32)
    bids = b_ref[0, 0, :]
    gids = lax.broadcasted_iota(jnp.int32, (NG, TA), 0)
    S = (gids == bids[None, :]).astype(jnp.float32)
    out_ref[...] += jnp.dot(S, e, preferred_element_type=jnp.float32)


def _energy(h, bids3, Wout_pad):
    return pl.pallas_call(
        _energy_body,
        grid=(NT,),
        in_specs=[
            pl.BlockSpec((TA, D_ATOM), lambda t: (t, 0)),
            pl.BlockSpec((1, 1, TA), lambda t: (t, 0, 0)),
            pl.BlockSpec((D_ATOM, 128), lambda t: (0, 0)),
        ],
        out_specs=pl.BlockSpec((NG, 128), lambda t: (0, 0)),
        out_shape=jax.ShapeDtypeStruct((NG, 128), jnp.float32),
    )(h, bids3, Wout_pad)


def kernel(atomic_numbers, pos, edge_index, batch_ids, emb_table, W_rbf,
           W_edge, W_m1, W_m2, W_gate, W_am, W_h, W_e, W_out):
    src = edge_index[0].astype(jnp.int32)
    dst = edge_index[1].astype(jnp.int32)
    perm = jnp.argsort(dst)
    dsts = dst[perm]
    srcs = src[perm]
    pad_e = EP - E
    zpad = jnp.zeros((pad_e,), jnp.int32)
    src_g = jnp.concatenate([srcs, zpad])
    dst_g = jnp.concatenate([dsts, zpad])
    dst_m = jnp.concatenate([dsts, jnp.full((pad_e,), 1 << 20, jnp.int32)])
    starts = jnp.searchsorted(
        dsts, jnp.arange(NT + 1, dtype=jnp.int32) * TA).astype(jnp.int32)

    an_pad = jnp.concatenate(
        [atomic_numbers.astype(jnp.int32), jnp.zeros((NPAD - N,), jnp.int32)])
    pos_pad = jnp.zeros((NPAD, 128), jnp.float32).at[:N, :3].set(pos)

    bf = jnp.bfloat16
    h = _sc_gather(emb_table, an_pad, 320)
    hp = jnp.concatenate([h, pos_pad], axis=1)
    hps = _sc_gather(hp, src_g, 256)
    hpd = _sc_gather(hp, dst_g, 256)

    m, rbf_emb = _edge_init(hps, hpd, W_rbf.astype(bf),
                            W_edge[:D_ATOM].astype(bf),
                            W_edge[D_ATOM:2 * D_ATOM].astype(bf),
                            W_edge[2 * D_ATOM:].astype(bf))

    for i in range(N_BLOCKS):
        mmid, amsg = _edge_mix(m, rbf_emb, W_gate[i].astype(bf),
                               W_m1[i].astype(bf), W_m2[i].astype(bf),
                               W_am[i].astype(bf))
        h, hd = _atom_update(starts, amsg, dst_m, h, W_h[i])
        hs = _sc_gather(h, src_g, 256)
        m = _edge_up(hs, hd, mmid, W_e[i][:D_ATOM].astype(bf),
                     W_e[i][D_ATOM:2 * D_ATOM].astype(bf),
                     W_e[i][2 * D_ATOM:].astype(bf))

    bids3 = jnp.concatenate(
        [batch_ids.astype(jnp.int32),
         jnp.full((NPAD - N,), NG, jnp.int32)]).reshape(NT, 1, TA)
    wout_pad = jnp.zeros((D_ATOM, 128), jnp.float32).at[:, :1].set(W_out)
    energy = _energy(h, bids3, wout_pad)
    return energy[:, 0]


# speedup vs baseline: 1.6973x; 1.4027x over previous
import functools

import jax
import jax.numpy as jnp
from jax import lax
from jax.experimental import pallas as pl
from jax.experimental.pallas import tpu as pltpu
from jax.experimental.pallas import tpu_sc as plsc

N = 10000
E = 160000
NG = 64
NUM_RADIAL = 128
EMB_RBF = 16
D_ATOM = 256
D_EDGE = 512
N_BLOCKS = 4
CUTOFF = 12.0

TA = 256
NT = 40
NPAD = TA * NT
EP = 163840
TE = 1024
CE = 1024
NW = 32
INV_SQRT2 = 0.7071067811865475


def _sc_gather(table, idx, chunk):
    row_shape = table.shape[1:]
    B = idx.shape[0]
    per_w = B // NW
    n_iter = per_w // chunk
    assert per_w % chunk == 0 and B % NW == 0 and per_w % 8 == 0

    def body(table_hbm, idx_hbm, out_hbm, idx_v, rows_v, sem):
        wid = lax.axis_index("s") * 2 + lax.axis_index("c")
        base = wid * per_w

        def step(j, carry):
            off = base + j * chunk
            pltpu.sync_copy(idx_hbm.at[pl.ds(off, chunk)], idx_v)
            pltpu.async_copy(table_hbm.at[idx_v], rows_v, sem).wait()
            pltpu.sync_copy(rows_v, out_hbm.at[pl.ds(off, chunk)])
            return carry

        lax.fori_loop(0, n_iter, step, 0)

    mesh = plsc.VectorSubcoreMesh(core_axis_name="c", subcore_axis_name="s")
    fn = pl.kernel(
        body,
        out_type=jax.ShapeDtypeStruct((B,) + row_shape, table.dtype),
        mesh=mesh,
        scratch_types=[
            pltpu.VMEM((chunk,), jnp.int32),
            pltpu.VMEM((chunk,) + row_shape, table.dtype),
            pltpu.SemaphoreType.DMA,
        ],
    )
    return fn(table, idx)


def _edge_init_body(hs_ref, hd_ref, wr_ref, ws_ref, wd_ref,
                    we_ref, m_ref, re_ref):
    vec = hd_ref[:, D_ATOM:] - hs_ref[:, D_ATOM:]
    d2 = jnp.sum(vec * vec, axis=1, keepdims=True)
    dist = jnp.sqrt(d2 + 1e-12) + 1e-6
    n = lax.broadcasted_iota(jnp.int32, (1, NUM_RADIAL), 1).astype(jnp.float32) + 1.0
    rbf = jnp.sqrt(2.0 / CUTOFF) * jnp.sin(n * (jnp.pi / CUTOFF) * dist) / dist
    u = jnp.clip(dist / CUTOFF, 0.0, 1.0)
    u5 = u * u * u * u * u
    env = 1.0 + (-21.0) * u5 + 35.0 * (u5 * u) + (-15.0) * (u5 * u * u)
    env = jnp.where(u < 1.0, env, 0.0)
    rbf = rbf * env
    re = jnp.dot(rbf.astype(jnp.bfloat16), wr_ref[...],
                 preferred_element_type=jnp.float32)
    z = (jnp.dot(hs_ref[:, :D_ATOM].astype(jnp.bfloat16), ws_ref[...],
                 preferred_element_type=jnp.float32)
         + jnp.dot(hd_ref[:, :D_ATOM].astype(jnp.bfloat16), wd_ref[...],
                   preferred_element_type=jnp.float32)
         + jnp.dot(re.astype(jnp.bfloat16), we_ref[...],
                   preferred_element_type=jnp.float32))
    m_ref[...] = (z * jax.nn.sigmoid(z)).astype(jnp.bfloat16)
    re_ref[...] = re


def _edge_init(hs, hd, W_rbf, We_s, We_d, We_r):
    grid = (EP // TE,)
    return pl.pallas_call(
        _edge_init_body,
        grid=grid,
        in_specs=[
            pl.BlockSpec((TE, D_ATOM + 128), lambda i: (i, 0)),
            pl.BlockSpec((TE, D_ATOM + 128), lambda i: (i, 0)),
            pl.BlockSpec((NUM_RADIAL, EMB_RBF), lambda i: (0, 0)),
            pl.BlockSpec((D_ATOM, D_EDGE), lambda i: (0, 0)),
            pl.BlockSpec((D_ATOM, D_EDGE), lambda i: (0, 0)),
            pl.BlockSpec((EMB_RBF, D_EDGE), lambda i: (0, 0)),
        ],
        out_specs=[
            pl.BlockSpec((TE, D_EDGE), lambda i: (i, 0)),
            pl.BlockSpec((TE, EMB_RBF), lambda i: (i, 0)),
        ],
        out_shape=[
            jax.ShapeDtypeStruct((EP, D_EDGE), jnp.bfloat16),
            jax.ShapeDtypeStruct((EP, EMB_RBF), jnp.float32),
        ],
    )(hs, hd, W_rbf, We_s, We_d, We_r)


def _edge_mix_body(m_ref, re_ref, wg_ref, w1_ref, w2_ref, wa_ref,
                   mmid_ref, amsg_ref):
    m0 = m_ref[...]
    gate = jnp.dot(re_ref[...].astype(jnp.bfloat16), wg_ref[...],
                   preferred_element_type=jnp.float32)
    z1 = jnp.dot(m0, w1_ref[...], preferred_element_type=jnp.float32)
    m2 = z1 * jax.nn.sigmoid(z1) * gate
    z2 = jnp.dot(m2.astype(jnp.bfloat16), w2_ref[...],
                 preferred_element_type=jnp.float32)
    m2 = z2 * jax.nn.sigmoid(z2)
    mm = (m0.astype(jnp.float32) + m2) * INV_SQRT2
    mmb = mm.astype(jnp.bfloat16)
    mmid_ref[...] = mmb
    amsg_ref[...] = jnp.dot(mmb, wa_ref[...],
                            preferred_element_type=jnp.float32).astype(jnp.bfloat16)


def _edge_mix(m, rbf_emb, Wgate, Wm1, Wm2, Wam):
    grid = (EP // TE,)
    return pl.pallas_call(
        _edge_mix_body,
        grid=grid,
        in_specs=[
            pl.BlockSpec((TE, D_EDGE), lambda i: (i, 0)),
            pl.BlockSpec((TE, EMB_RBF), lambda i: (i, 0)),
            pl.BlockSpec((EMB_RBF, D_EDGE), lambda i: (0, 0)),
            pl.BlockSpec((D_EDGE, D_EDGE), lambda i: (0, 0)),
            pl.BlockSpec((D_EDGE, D_EDGE), lambda i: (0, 0)),
            pl.BlockSpec((D_EDGE, D_ATOM), lambda i: (0, 0)),
        ],
        out_specs=[
            pl.BlockSpec((TE, D_EDGE), lambda i: (i, 0)),
            pl.BlockSpec((TE, D_ATOM), lambda i: (i, 0)),
        ],
        out_shape=[
            jax.ShapeDtypeStruct((EP, D_EDGE), jnp.bfloat16),
            jax.ShapeDtypeStruct((EP, D_ATOM), jnp.bfloat16),
        ],
    )(m, rbf_emb, Wgate, Wm1, Wm2, Wam)


def _atom_body(starts_ref, amsg_hbm, dst_hbm, h_ref, wh_ref, hnew_ref,
               hdst_hbm, am0, am1, ds0, ds1, agg_ref, hd0, hd1,
               sema0, sema1, semd0, semd1, semr, semw0, semw1):
    t = pl.program_id(0)
    start = starts_ref[t]
    end = starts_ref[t + 1]
    end = jnp.where(t == NT - 1, EP, end)
    c0 = start // CE
    nch = jnp.maximum(0, (end + CE - 1) // CE - c0)
    agg_ref[...] = jnp.zeros((TA, D_ATOM), jnp.float32)

    def issue(j, am, ds, sa, sd):
        off = pl.multiple_of((c0 + j) * CE, CE)
        pltpu.make_async_copy(amsg_hbm.at[pl.ds(off, CE)], am, sa).start()
        pltpu.make_async_copy(dst_hbm.at[pl.ds(off, CE)], ds, sd).start()

    def wait_in(am, ds, sa, sd):
        pltpu.make_async_copy(amsg_hbm.at[pl.ds(0, CE)], am, sa).wait()
        pltpu.make_async_copy(dst_hbm.at[pl.ds(0, CE)], ds, sd).wait()

    def process(am, ds):
        dstl = ds[...] - t * TA
        ids = lax.broadcasted_iota(jnp.int32, (TA, CE), 0)
        S = (ids == dstl[None, :]).astype(jnp.bfloat16)
        agg_ref[...] += jnp.dot(S, am[...], preferred_element_type=jnp.float32)

    @pl.when(nch > 0)
    def _():
        issue(0, am0, ds0, sema0, semd0)

    def pair(k, carry):
        j1 = 2 * k + 1

        @pl.when(j1 < nch)
        def _():
            issue(j1, am1, ds1, sema1, semd1)

        wait_in(am0, ds0, sema0, semd0)
        process(am0, ds0)

        @pl.when(j1 < nch)
        def _():
            @pl.when(j1 + 1 < nch)
            def _():
                issue(j1 + 1, am0, ds0, sema0, semd0)

            wait_in(am1, ds1, sema1, semd1)
            process(am1, ds1)

        return carry

    lax.fori_loop(0, (nch + 1) // 2, pair, 0)
    z = jnp.dot(agg_ref[...], wh_ref[...], preferred_element_type=jnp.float32)
    hnew = h_ref[...] + z * jax.nn.sigmoid(z)
    hnew_ref[...] = hnew
    hb = hnew.astype(jnp.bfloat16)

    def issue_d(j, ds, sd):
        off = pl.multiple_of((c0 + j) * CE, CE)
        pltpu.make_async_copy(dst_hbm.at[pl.ds(off, CE)], ds, sd).start()

    def wait_d(ds, sd):
        pltpu.make_async_copy(dst_hbm.at[pl.ds(0, CE)], ds, sd).wait()

    def expand_write(j, ds, hdbuf, semw):
        off = pl.multiple_of((c0 + j) * CE, CE)
        dstl = ds[...] - t * TA
        ids2 = lax.broadcasted_iota(jnp.int32, (CE, TA), 1)
        G = (ids2 == dstl[:, None]).astype(jnp.bfloat16)
        hd = jnp.dot(G, hb, preferred_element_type=jnp.float32).astype(jnp.bfloat16)

        @pl.when(j >= 2)
        def _():
            pltpu.make_async_copy(hdbuf, hdst_hbm.at[pl.ds(0, CE)], semw).wait()

        shared = jnp.logical_and(j == 0, start % CE != 0)

        @pl.when(shared)
        def _():
            cr = pltpu.make_async_copy(hdst_hbm.at[pl.ds(off, CE)], hdbuf, semr)
            cr.start()
            cr.wait()
            hdbuf[...] = hdbuf[...] + hd

        @pl.when(jnp.logical_not(shared))
        def _():
            hdbuf[...] = hd

        pltpu.make_async_copy(hdbuf, hdst_hbm.at[pl.ds(off, CE)], semw).start()

    @pl.when(nch > 0)
    def _():
        issue_d(0, ds0, semd0)

    def pair2(k, carry):
        j1 = 2 * k + 1

        @pl.when(j1 < nch)
        def _():
            issue_d(j1, ds1, semd1)

        wait_d(ds0, semd0)
        expand_write(2 * k, ds0, hd0, semw0)

        @pl.when(j1 < nch)
        def _():
            @pl.when(j1 + 1 < nch)
            def _():
                issue_d(j1 + 1, ds0, semd0)

            wait_d(ds1, semd1)
            expand_write(j1, ds1, hd1, semw1)

        return carry

    lax.fori_loop(0, (nch + 1) // 2, pair2, 0)

    @pl.when(nch >= 1)
    def _():
        pltpu.make_async_copy(hd0, hdst_hbm.at[pl.ds(0, CE)], semw0).wait()

    @pl.when(nch >= 2)
    def _():
        pltpu.make_async_copy(hd1, hdst_hbm.at[pl.ds(0, CE)], semw1).wait()


def _atom_update(starts, amsg, dst_m, h, Wh):
    grid_spec = pltpu.PrefetchScalarGridSpec(
        num_scalar_prefetch=1,
        grid=(NT,),
        in_specs=[
            pl.BlockSpec(memory_space=pl.ANY),
            pl.BlockSpec(memory_space=pl.ANY),
            pl.BlockSpec((TA, D_ATOM), lambda t, starts: (t, 0)),
            pl.BlockSpec((D_ATOM, D_ATOM), lambda t, starts: (0, 0)),
        ],
        out_specs=[
            pl.BlockSpec((TA, D_ATOM), lambda t, starts: (t, 0)),
            pl.BlockSpec(memory_space=pl.ANY),
        ],
        scratch_shapes=[
            pltpu.VMEM((CE, D_ATOM), jnp.bfloat16),
            pltpu.VMEM((CE, D_ATOM), jnp.bfloat16),
            pltpu.VMEM((CE,), jnp.int32),
            pltpu.VMEM((CE,), jnp.int32),
            pltpu.VMEM((TA, D_ATOM), jnp.float32),
            pltpu.VMEM((CE, D_ATOM), jnp.bfloat16),
            pltpu.VMEM((CE, D_ATOM), jnp.bfloat16),
            pltpu.SemaphoreType.DMA,
            pltpu.SemaphoreType.DMA,
            pltpu.SemaphoreType.DMA,
            pltpu.SemaphoreType.DMA,
            pltpu.SemaphoreType.DMA,
            pltpu.SemaphoreType.DMA,
            pltpu.SemaphoreType.DMA,
        ],
    )
    return pl.pallas_call(
        _atom_body,
        grid_spec=grid_spec,
        out_shape=[
            jax.ShapeDtypeStruct((NPAD, D_ATOM), jnp.float32),
            jax.ShapeDtypeStruct((EP, D_ATOM), jnp.bfloat16),
        ],
    )(starts, amsg, dst_m, h, Wh)


def _edge_up_body(hs_ref, hd_ref, m_ref, ws_ref, wd_ref, wm_ref, out_ref):
    z = (jnp.dot(hs_ref[...].astype(jnp.bfloat16), ws_ref[...],
                 preferred_element_type=jnp.float32)
         + jnp.dot(hd_ref[...], wd_ref[...], preferred_element_type=jnp.float32)
         + jnp.dot(m_ref[...], wm_ref[...], preferred_element_type=jnp.float32))
    e = z * jax.nn.sigmoid(z)
    out_ref[...] = ((m_ref[...].astype(jnp.float32) + e)
                    * INV_SQRT2).astype(jnp.bfloat16)


def _edge_up(hs, hd, mmid, We_s, We_d, We_m):
    grid = (EP // TE,)
    return pl.pallas_call(
        _edge_up_body,
        grid=grid,
        in_specs=[
            pl.BlockSpec((TE, D_ATOM), lambda i: (i, 0)),
            pl.BlockSpec((TE, D_ATOM), lambda i: (i, 0)),
            pl.BlockSpec((TE, D_EDGE), lambda i: (i, 0)),
            pl.BlockSpec((D_ATOM, D_EDGE), lambda i: (0, 0)),
            pl.BlockSpec((D_ATOM, D_EDGE), lambda i: (0, 0)),
            pl.BlockSpec((D_EDGE, D_EDGE), lambda i: (0, 0)),
        ],
        out_specs=pl.BlockSpec((TE, D_EDGE), lambda i: (i, 0)),
        out_shape=jax.ShapeDtypeStruct((EP, D_EDGE), jnp.bfloat16),
    )(hs, hd, mmid, We_s, We_d, We_m)


def _energy_body(h_ref, b_ref, w_ref, out_ref):
    t = pl.program_id(0)

    @pl.when(t == 0)
    def _():
        out_ref[...] = jnp.zeros_like(out_ref)

    e = jnp.dot(h_ref[...], w_ref[...], preferred_element_type=jnp.float32)
    bids = b_ref[0, 0, :]
    gids = lax.broadcasted_iota(jnp.int32, (NG, TA), 0)
    S = (gids == bids[None, :]).astype(jnp.float32)
    out_ref[...] += jnp.dot(S, e, preferred_element_type=jnp.float32)


def _energy(h, bids3, Wout_pad):
    return pl.pallas_call(
        _energy_body,
        grid=(NT,),
        in_specs=[
            pl.BlockSpec((TA, D_ATOM), lambda t: (t, 0)),
            pl.BlockSpec((1, 1, TA), lambda t: (t, 0, 0)),
            pl.BlockSpec((D_ATOM, 128), lambda t: (0, 0)),
        ],
        out_specs=pl.BlockSpec((NG, 128), lambda t: (0, 0)),
        out_shape=jax.ShapeDtypeStruct((NG, 128), jnp.float32),
    )(h, bids3, Wout_pad)


def kernel(atomic_numbers, pos, edge_index, batch_ids, emb_table, W_rbf,
           W_edge, W_m1, W_m2, W_gate, W_am, W_h, W_e, W_out):
    src = edge_index[0].astype(jnp.int32)
    dst = edge_index[1].astype(jnp.int32)
    perm = jnp.argsort(dst)
    dsts = dst[perm]
    srcs = src[perm]
    pad_e = EP - E
    zpad = jnp.zeros((pad_e,), jnp.int32)
    src_g = jnp.concatenate([srcs, zpad])
    dst_g = jnp.concatenate([dsts, zpad])
    dst_m = jnp.concatenate([dsts, jnp.full((pad_e,), 1 << 20, jnp.int32)])
    starts = jnp.searchsorted(
        dsts, jnp.arange(NT + 1, dtype=jnp.int32) * TA).astype(jnp.int32)

    an_pad = jnp.concatenate(
        [atomic_numbers.astype(jnp.int32), jnp.zeros((NPAD - N,), jnp.int32)])
    pos_pad = jnp.zeros((NPAD, 128), jnp.float32).at[:N, :3].set(pos)

    bf = jnp.bfloat16
    h = _sc_gather(emb_table, an_pad, 320)
    hp = jnp.concatenate([h, pos_pad], axis=1)
    hps = _sc_gather(hp, src_g, 256)
    hpd = _sc_gather(hp, dst_g, 256)

    m, rbf_emb = _edge_init(hps, hpd, W_rbf.astype(bf),
                            W_edge[:D_ATOM].astype(bf),
                            W_edge[D_ATOM:2 * D_ATOM].astype(bf),
                            W_edge[2 * D_ATOM:].astype(bf))

    for i in range(N_BLOCKS):
        mmid, amsg = _edge_mix(m, rbf_emb, W_gate[i].astype(bf),
                               W_m1[i].astype(bf), W_m2[i].astype(bf),
                               W_am[i].astype(bf))
        h, hd = _atom_update(starts, amsg, dst_m, h, W_h[i])
        hs = _sc_gather(h, src_g, 256)
        m = _edge_up(hs, hd, mmid, W_e[i][:D_ATOM].astype(bf),
                     W_e[i][D_ATOM:2 * D_ATOM].astype(bf),
                     W_e[i][2 * D_ATOM:].astype(bf))

    bids3 = jnp.concatenate(
        [batch_ids.astype(jnp.int32),
         jnp.full((NPAD - N,), NG, jnp.int32)]).reshape(NT, 1, TA)
    wout_pad = jnp.zeros((D_ATOM, 128), jnp.float32).at[:, :1].set(W_out)
    energy = _energy(h, bids3, wout_pad)
    return energy[:, 0]


# speedup vs baseline: 1.7519x; 1.0322x over previous
import functools

import jax
import jax.numpy as jnp
from jax import lax
from jax.experimental import pallas as pl
from jax.experimental.pallas import tpu as pltpu
from jax.experimental.pallas import tpu_sc as plsc

N = 10000
E = 160000
NG = 64
NUM_RADIAL = 128
EMB_RBF = 16
D_ATOM = 256
D_EDGE = 512
N_BLOCKS = 4
CUTOFF = 12.0

TA = 256
NT = 40
NPAD = TA * NT
EP = 163840
TE = 1024
CE = 1024
NW = 32
INV_SQRT2 = 0.7071067811865475


def _sc_gather(table, idx, chunk):
    row_shape = table.shape[1:]
    B = idx.shape[0]
    per_w = B // NW
    n_iter = per_w // chunk
    npair = n_iter // 2
    assert per_w % chunk == 0 and B % NW == 0 and chunk % 8 == 0
    assert n_iter % 2 == 0 and n_iter >= 2

    def body(table_hbm, idx_hbm, out_hbm, idx_all, r0, r1, gs0, gs1, ws0, ws1):
        wid = lax.axis_index("s") * 2 + lax.axis_index("c")
        base = wid * per_w
        pltpu.sync_copy(idx_hbm.at[pl.ds(base, per_w)], idx_all)

        def g(j, buf, sem):
            pltpu.make_async_copy(
                table_hbm.at[idx_all.at[pl.ds(j * chunk, chunk)]], buf,
                sem).start()

        def w(j, buf, sem):
            pltpu.make_async_copy(
                buf, out_hbm.at[pl.ds(base + j * chunk, chunk)], sem).start()

        def wait_g(buf, sem):
            pltpu.make_async_copy(
                table_hbm.at[idx_all.at[pl.ds(0, chunk)]], buf, sem).wait()

        def wait_w(buf, sem):
            pltpu.make_async_copy(
                buf, out_hbm.at[pl.ds(base, chunk)], sem).wait()

        g(0, r0, gs0)
        g(1, r1, gs1)

        def pair(k, carry):
            j0 = 2 * k
            wait_g(r0, gs0)
            w(j0, r0, ws0)
            wait_g(r1, gs1)
            w(j0 + 1, r1, ws1)

            @pl.when(k + 1 < npair)
            def _():
                wait_w(r0, ws0)
                g(j0 + 2, r0, gs0)
                wait_w(r1, ws1)
                g(j0 + 3, r1, gs1)

            return carry

        lax.fori_loop(0, npair, pair, 0)
        wait_w(r0, ws0)
        wait_w(r1, ws1)

    mesh = plsc.VectorSubcoreMesh(core_axis_name="c", subcore_axis_name="s")
    fn = pl.kernel(
        body,
        out_type=jax.ShapeDtypeStruct((B,) + row_shape, table.dtype),
        mesh=mesh,
        scratch_types=[
            pltpu.VMEM((per_w,), jnp.int32),
            pltpu.VMEM((chunk,) + row_shape, table.dtype),
            pltpu.VMEM((chunk,) + row_shape, table.dtype),
            pltpu.SemaphoreType.DMA,
            pltpu.SemaphoreType.DMA,
            pltpu.SemaphoreType.DMA,
            pltpu.SemaphoreType.DMA,
        ],
    )
    return fn(table, idx)


def _edge_init_body(hs_ref, hd_ref, wr_ref, ws_ref, wd_ref,
                    we_ref, m_ref, re_ref):
    vec = hd_ref[:, D_ATOM:] - hs_ref[:, D_ATOM:]
    d2 = jnp.sum(vec * vec, axis=1, keepdims=True)
    dist = jnp.sqrt(d2 + 1e-12) + 1e-6
    n = lax.broadcasted_iota(jnp.int32, (1, NUM_RADIAL), 1).astype(jnp.float32) + 1.0
    rbf = jnp.sqrt(2.0 / CUTOFF) * jnp.sin(n * (jnp.pi / CUTOFF) * dist) / dist
    u = jnp.clip(dist / CUTOFF, 0.0, 1.0)
    u5 = u * u * u * u * u
    env = 1.0 + (-21.0) * u5 + 35.0 * (u5 * u) + (-15.0) * (u5 * u * u)
    env = jnp.where(u < 1.0, env, 0.0)
    rbf = rbf * env
    re = jnp.dot(rbf.astype(jnp.bfloat16), wr_ref[...],
                 preferred_element_type=jnp.float32)
    z = (jnp.dot(hs_ref[:, :D_ATOM].astype(jnp.bfloat16), ws_ref[...],
                 preferred_element_type=jnp.float32)
         + jnp.dot(hd_ref[:, :D_ATOM].astype(jnp.bfloat16), wd_ref[...],
                   preferred_element_type=jnp.float32)
         + jnp.dot(re.astype(jnp.bfloat16), we_ref[...],
                   preferred_element_type=jnp.float32))
    m_ref[...] = (z * jax.nn.sigmoid(z)).astype(jnp.bfloat16)
    re_ref[...] = re


def _edge_init(hs, hd, W_rbf, We_s, We_d, We_r):
    grid = (EP // TE,)
    return pl.pallas_call(
        _edge_init_body,
        grid=grid,
        in_specs=[
            pl.BlockSpec((TE, D_ATOM + 128), lambda i: (i, 0)),
            pl.BlockSpec((TE, D_ATOM + 128), lambda i: (i, 0)),
            pl.BlockSpec((NUM_RADIAL, EMB_RBF), lambda i: (0, 0)),
            pl.BlockSpec((D_ATOM, D_EDGE), lambda i: (0, 0)),
            pl.BlockSpec((D_ATOM, D_EDGE), lambda i: (0, 0)),
            pl.BlockSpec((EMB_RBF, D_EDGE), lambda i: (0, 0)),
        ],
        out_specs=[
            pl.BlockSpec((TE, D_EDGE), lambda i: (i, 0)),
            pl.BlockSpec((TE, EMB_RBF), lambda i: (i, 0)),
        ],
        out_shape=[
            jax.ShapeDtypeStruct((EP, D_EDGE), jnp.bfloat16),
            jax.ShapeDtypeStruct((EP, EMB_RBF), jnp.float32),
        ],
    )(hs, hd, W_rbf, We_s, We_d, We_r)


def _edge_mix_body(m_ref, re_ref, wg_ref, w1_ref, w2_ref, wa_ref,
                   mmid_ref, amsg_ref):
    m0 = m_ref[...]
    gate = jnp.dot(re_ref[...].astype(jnp.bfloat16), wg_ref[...],
                   preferred_element_type=jnp.float32)
    z1 = jnp.dot(m0, w1_ref[...], preferred_element_type=jnp.float32)
    m2 = z1 * jax.nn.sigmoid(z1) * gate
    z2 = jnp.dot(m2.astype(jnp.bfloat16), w2_ref[...],
                 preferred_element_type=jnp.float32)
    m2 = z2 * jax.nn.sigmoid(z2)
    mm = (m0.astype(jnp.float32) + m2) * INV_SQRT2
    mmb = mm.astype(jnp.bfloat16)
    mmid_ref[...] = mmb
    amsg_ref[...] = jnp.dot(mmb, wa_ref[...],
                            preferred_element_type=jnp.float32).astype(jnp.bfloat16)


def _edge_mix(m, rbf_emb, Wgate, Wm1, Wm2, Wam):
    grid = (EP // TE,)
    return pl.pallas_call(
        _edge_mix_body,
        grid=grid,
        in_specs=[
            pl.BlockSpec((TE, D_EDGE), lambda i: (i, 0)),
            pl.BlockSpec((TE, EMB_RBF), lambda i: (i, 0)),
            pl.BlockSpec((EMB_RBF, D_EDGE), lambda i: (0, 0)),
            pl.BlockSpec((D_EDGE, D_EDGE), lambda i: (0, 0)),
            pl.BlockSpec((D_EDGE, D_EDGE), lambda i: (0, 0)),
            pl.BlockSpec((D_EDGE, D_ATOM), lambda i: (0, 0)),
        ],
        out_specs=[
            pl.BlockSpec((TE, D_EDGE), lambda i: (i, 0)),
            pl.BlockSpec((TE, D_ATOM), lambda i: (i, 0)),
        ],
        out_shape=[
            jax.ShapeDtypeStruct((EP, D_EDGE), jnp.bfloat16),
            jax.ShapeDtypeStruct((EP, D_ATOM), jnp.bfloat16),
        ],
    )(m, rbf_emb, Wgate, Wm1, Wm2, Wam)


def _atom_body(starts_ref, amsg_hbm, dst_hbm, h_ref, wh_ref, hnew_ref,
               hdst_hbm, am0, am1, ds0, ds1, agg_ref, hd0, hd1,
               sema0, sema1, semd0, semd1, semr, semw0, semw1):
    t = pl.program_id(0)
    start = starts_ref[t]
    end = starts_ref[t + 1]
    end = jnp.where(t == NT - 1, EP, end)
    c0 = start // CE
    nch = jnp.maximum(0, (end + CE - 1) // CE - c0)
    agg_ref[...] = jnp.zeros((TA, D_ATOM), jnp.float32)

    def issue(j, am, ds, sa, sd):
        off = pl.multiple_of((c0 + j) * CE, CE)
        pltpu.make_async_copy(amsg_hbm.at[pl.ds(off, CE)], am, sa).start()
        pltpu.make_async_copy(dst_hbm.at[pl.ds(off, CE)], ds, sd).start()

    def wait_in(am, ds, sa, sd):
        pltpu.make_async_copy(amsg_hbm.at[pl.ds(0, CE)], am, sa).wait()
        pltpu.make_async_copy(dst_hbm.at[pl.ds(0, CE)], ds, sd).wait()

    def process(am, ds):
        dstl = ds[...] - t * TA
        ids = lax.broadcasted_iota(jnp.int32, (TA, CE), 0)
        S = (ids == dstl[None, :]).astype(jnp.bfloat16)
        agg_ref[...] += jnp.dot(S, am[...], preferred_element_type=jnp.float32)

    @pl.when(nch > 0)
    def _():
        issue(0, am0, ds0, sema0, semd0)

    def pair(k, carry):
        j1 = 2 * k + 1

        @pl.when(j1 < nch)
        def _():
            issue(j1, am1, ds1, sema1, semd1)

        wait_in(am0, ds0, sema0, semd0)
        process(am0, ds0)

        @pl.when(j1 < nch)
        def _():
            @pl.when(j1 + 1 < nch)
            def _():
                issue(j1 + 1, am0, ds0, sema0, semd0)

            wait_in(am1, ds1, sema1, semd1)
            process(am1, ds1)

        return carry

    lax.fori_loop(0, (nch + 1) // 2, pair, 0)
    z = jnp.dot(agg_ref[...], wh_ref[...], preferred_element_type=jnp.float32)
    hnew = h_ref[...] + z * jax.nn.sigmoid(z)
    hnew_ref[...] = hnew
    hb = hnew.astype(jnp.bfloat16)

    def issue_d(j, ds, sd):
        off = pl.multiple_of((c0 + j) * CE, CE)
        pltpu.make_async_copy(dst_hbm.at[pl.ds(off, CE)], ds, sd).start()

    def wait_d(ds, sd):
        pltpu.make_async_copy(dst_hbm.at[pl.ds(0, CE)], ds, sd).wait()

    def expand_write(j, ds, hdbuf, semw):
        off = pl.multiple_of((c0 + j) * CE, CE)
        dstl = ds[...] - t * TA
        ids2 = lax.broadcasted_iota(jnp.int32, (CE, TA), 1)
        G = (ids2 == dstl[:, None]).astype(jnp.bfloat16)
        hd = jnp.dot(G, hb, preferred_element_type=jnp.float32).astype(jnp.bfloat16)

        @pl.when(j >= 2)
        def _():
            pltpu.make_async_copy(hdbuf, hdst_hbm.at[pl.ds(0, CE)], semw).wait()

        shared = jnp.logical_and(j == 0, start % CE != 0)

        @pl.when(shared)
        def _():
            cr = pltpu.make_async_copy(hdst_hbm.at[pl.ds(off, CE)], hdbuf, semr)
            cr.start()
            cr.wait()
            hdbuf[...] = hdbuf[...] + hd

        @pl.when(jnp.logical_not(shared))
        def _():
            hdbuf[...] = hd

        pltpu.make_async_copy(hdbuf, hdst_hbm.at[pl.ds(off, CE)], semw).start()

    @pl.when(nch > 0)
    def _():
        issue_d(0, ds0, semd0)

    def pair2(k, carry):
        j1 = 2 * k + 1

        @pl.when(j1 < nch)
        def _():
            issue_d(j1, ds1, semd1)

        wait_d(ds0, semd0)
        expand_write(2 * k, ds0, hd0, semw0)

        @pl.when(j1 < nch)
        def _():
            @pl.when(j1 + 1 < nch)
            def _():
                issue_d(j1 + 1, ds0, semd0)

            wait_d(ds1, semd1)
            expand_write(j1, ds1, hd1, semw1)

        return carry

    lax.fori_loop(0, (nch + 1) // 2, pair2, 0)

    @pl.when(nch >= 1)
    def _():
        pltpu.make_async_copy(hd0, hdst_hbm.at[pl.ds(0, CE)], semw0).wait()

    @pl.when(nch >= 2)
    def _():
        pltpu.make_async_copy(hd1, hdst_hbm.at[pl.ds(0, CE)], semw1).wait()


def _atom_update(starts, amsg, dst_m, h, Wh):
    grid_spec = pltpu.PrefetchScalarGridSpec(
        num_scalar_prefetch=1,
        grid=(NT,),
        in_specs=[
            pl.BlockSpec(memory_space=pl.ANY),
            pl.BlockSpec(memory_space=pl.ANY),
            pl.BlockSpec((TA, D_ATOM), lambda t, starts: (t, 0)),
            pl.BlockSpec((D_ATOM, D_ATOM), lambda t, starts: (0, 0)),
        ],
        out_specs=[
            pl.BlockSpec((TA, D_ATOM), lambda t, starts: (t, 0)),
            pl.BlockSpec(memory_space=pl.ANY),
        ],
        scratch_shapes=[
            pltpu.VMEM((CE, D_ATOM), jnp.bfloat16),
            pltpu.VMEM((CE, D_ATOM), jnp.bfloat16),
            pltpu.VMEM((CE,), jnp.int32),
            pltpu.VMEM((CE,), jnp.int32),
            pltpu.VMEM((TA, D_ATOM), jnp.float32),
            pltpu.VMEM((CE, D_ATOM), jnp.bfloat16),
            pltpu.VMEM((CE, D_ATOM), jnp.bfloat16),
            pltpu.SemaphoreType.DMA,
            pltpu.SemaphoreType.DMA,
            pltpu.SemaphoreType.DMA,
            pltpu.SemaphoreType.DMA,
            pltpu.SemaphoreType.DMA,
            pltpu.SemaphoreType.DMA,
            pltpu.SemaphoreType.DMA,
        ],
    )
    return pl.pallas_call(
        _atom_body,
        grid_spec=grid_spec,
        out_shape=[
            jax.ShapeDtypeStruct((NPAD, D_ATOM), jnp.float32),
            jax.ShapeDtypeStruct((EP, D_ATOM), jnp.bfloat16),
        ],
    )(starts, amsg, dst_m, h, Wh)


def _edge_up_body(hs_ref, hd_ref, m_ref, ws_ref, wd_ref, wm_ref, out_ref):
    z = (jnp.dot(hs_ref[...].astype(jnp.bfloat16), ws_ref[...],
                 preferred_element_type=jnp.float32)
         + jnp.dot(hd_ref[...], wd_ref[...], preferred_element_type=jnp.float32)
         + jnp.dot(m_ref[...], wm_ref[...], preferred_element_type=jnp.float32))
    e = z * jax.nn.sigmoid(z)
    out_ref[...] = ((m_ref[...].astype(jnp.float32) + e)
                    * INV_SQRT2).astype(jnp.bfloat16)


def _edge_up(hs, hd, mmid, We_s, We_d, We_m):
    grid = (EP // TE,)
    return pl.pallas_call(
        _edge_up_body,
        grid=grid,
        in_specs=[
            pl.BlockSpec((TE, D_ATOM), lambda i: (i, 0)),
            pl.BlockSpec((TE, D_ATOM), lambda i: (i, 0)),
            pl.BlockSpec((TE, D_EDGE), lambda i: (i, 0)),
            pl.BlockSpec((D_ATOM, D_EDGE), lambda i: (0, 0)),
            pl.BlockSpec((D_ATOM, D_EDGE), lambda i: (0, 0)),
            pl.BlockSpec((D_EDGE, D_EDGE), lambda i: (0, 0)),
        ],
        out_specs=pl.BlockSpec((TE, D_EDGE), lambda i: (i, 0)),
        out_shape=jax.ShapeDtypeStruct((EP, D_EDGE), jnp.bfloat16),
    )(hs, hd, mmid, We_s, We_d, We_m)


def _energy_body(h_ref, b_ref, w_ref, out_ref):
    t = pl.program_id(0)

    @pl.when(t == 0)
    def _():
        out_ref[...] = jnp.zeros_like(out_ref)

    e = jnp.dot(h_ref[...], w_ref[...], preferred_element_type=jnp.float32)
    bids = b_ref[0, 0, :]
    gids = lax.broadcasted_iota(jnp.int32, (NG, TA), 0)
    S = (gids == bids[None, :]).astype(jnp.float32)
    out_ref[...] += jnp.dot(S, e, preferred_element_type=jnp.float32)


def _energy(h, bids3, Wout_pad):
    return pl.pallas_call(
        _energy_body,
        grid=(NT,),
        in_specs=[
            pl.BlockSpec((TA, D_ATOM), lambda t: (t, 0)),
            pl.BlockSpec((1, 1, TA), lambda t: (t, 0, 0)),
            pl.BlockSpec((D_ATOM, 128), lambda t: (0, 0)),
        ],
        out_specs=pl.BlockSpec((NG, 128), lambda t: (0, 0)),
        out_shape=jax.ShapeDtypeStruct((NG, 128), jnp.float32),
    )(h, bids3, Wout_pad)


def kernel(atomic_numbers, pos, edge_index, batch_ids, emb_table, W_rbf,
           W_edge, W_m1, W_m2, W_gate, W_am, W_h, W_e, W_out):
    src = edge_index[0].astype(jnp.int32)
    dst = edge_index[1].astype(jnp.int32)
    perm = jnp.argsort(dst)
    dsts = dst[perm]
    srcs = src[perm]
    pad_e = EP - E
    zpad = jnp.zeros((pad_e,), jnp.int32)
    src_g = jnp.concatenate([srcs, zpad])
    dst_g = jnp.concatenate([dsts, zpad])
    dst_m = jnp.concatenate([dsts, jnp.full((pad_e,), 1 << 20, jnp.int32)])
    starts = jnp.searchsorted(
        dsts, jnp.arange(NT + 1, dtype=jnp.int32) * TA).astype(jnp.int32)

    an_pad = jnp.concatenate(
        [atomic_numbers.astype(jnp.int32), jnp.zeros((NPAD - N,), jnp.int32)])
    pos_pad = jnp.zeros((NPAD, 128), jnp.float32).at[:N, :3].set(pos)

    bf = jnp.bfloat16
    h = _sc_gather(emb_table, an_pad, 160)
    hp = jnp.concatenate([h, pos_pad], axis=1)
    hps = _sc_gather(hp, src_g, 160)
    hpd = _sc_gather(hp, dst_g, 160)

    m, rbf_emb = _edge_init(hps, hpd, W_rbf.astype(bf),
                            W_edge[:D_ATOM].astype(bf),
                            W_edge[D_ATOM:2 * D_ATOM].astype(bf),
                            W_edge[2 * D_ATOM:].astype(bf))

    for i in range(N_BLOCKS):
        mmid, amsg = _edge_mix(m, rbf_emb, W_gate[i].astype(bf),
                               W_m1[i].astype(bf), W_m2[i].astype(bf),
                               W_am[i].astype(bf))
        h, hd = _atom_update(starts, amsg, dst_m, h, W_h[i])
        hs = _sc_gather(h, src_g, 160)
        m = _edge_up(hs, hd, mmid, W_e[i][:D_ATOM].astype(bf),
                     W_e[i][D_ATOM:2 * D_ATOM].astype(bf),
                     W_e[i][2 * D_ATOM:].astype(bf))

    bids3 = jnp.concatenate(
        [batch_ids.astype(jnp.int32),
         jnp.full((NPAD - N,), NG, jnp.int32)]).reshape(NT, 1, TA)
    wout_pad = jnp.zeros((D_ATOM, 128), jnp.float32).at[:, :1].set(W_out)
    energy = _energy(h, bids3, wout_pad)
    return energy[:, 0]


# speedup vs baseline: 1.9364x; 1.1053x over previous
import functools

import jax
import jax.numpy as jnp
from jax import lax
from jax.experimental import pallas as pl
from jax.experimental.pallas import tpu as pltpu
from jax.experimental.pallas import tpu_sc as plsc

N = 10000
E = 160000
NG = 64
NUM_RADIAL = 128
EMB_RBF = 16
D_ATOM = 256
D_EDGE = 512
N_BLOCKS = 4
CUTOFF = 12.0

TA = 256
NT = 40
NPAD = TA * NT
EP = 163840
TE = 1024
CE = 1024
NW = 32
INV_SQRT2 = 0.7071067811865475


def _sc_gather(table, idx, chunk):
    row_shape = table.shape[1:]
    B = idx.shape[0]
    per_w = B // NW
    n_iter = per_w // chunk
    npair = n_iter // 2
    assert per_w % chunk == 0 and B % NW == 0 and chunk % 8 == 0
    assert n_iter % 2 == 0 and n_iter >= 2

    def body(table_hbm, idx_hbm, out_hbm, idx_all, r0, r1, gs0, gs1, ws0, ws1):
        wid = lax.axis_index("s") * 2 + lax.axis_index("c")
        base = wid * per_w
        pltpu.sync_copy(idx_hbm.at[pl.ds(base, per_w)], idx_all)

        def g(j, buf, sem):
            pltpu.make_async_copy(
                table_hbm.at[idx_all.at[pl.ds(j * chunk, chunk)]], buf,
                sem).start()

        def w(j, buf, sem):
            pltpu.make_async_copy(
                buf, out_hbm.at[pl.ds(base + j * chunk, chunk)], sem).start()

        def wait_g(buf, sem):
            pltpu.make_async_copy(
                table_hbm.at[idx_all.at[pl.ds(0, chunk)]], buf, sem).wait()

        def wait_w(buf, sem):
            pltpu.make_async_copy(
                buf, out_hbm.at[pl.ds(base, chunk)], sem).wait()

        g(0, r0, gs0)
        g(1, r1, gs1)

        def pair(k, carry):
            j0 = 2 * k
            wait_g(r0, gs0)
            w(j0, r0, ws0)
            wait_g(r1, gs1)
            w(j0 + 1, r1, ws1)

            @pl.when(k + 1 < npair)
            def _():
                wait_w(r0, ws0)
                g(j0 + 2, r0, gs0)
                wait_w(r1, ws1)
                g(j0 + 3, r1, gs1)

            return carry

        lax.fori_loop(0, npair, pair, 0)
        wait_w(r0, ws0)
        wait_w(r1, ws1)

    mesh = plsc.VectorSubcoreMesh(core_axis_name="c", subcore_axis_name="s")
    fn = pl.kernel(
        body,
        out_type=jax.ShapeDtypeStruct((B,) + row_shape, table.dtype),
        mesh=mesh,
        scratch_types=[
            pltpu.VMEM((per_w,), jnp.int32),
            pltpu.VMEM((chunk,) + row_shape, table.dtype),
            pltpu.VMEM((chunk,) + row_shape, table.dtype),
            pltpu.SemaphoreType.DMA,
            pltpu.SemaphoreType.DMA,
            pltpu.SemaphoreType.DMA,
            pltpu.SemaphoreType.DMA,
        ],
    )
    return fn(table, idx)


def _edge_init_body(hs_ref, hd_ref, wr_ref, ws_ref, wd_ref,
                    we_ref, m_ref, re_ref):
    vec = hd_ref[:, D_ATOM:] - hs_ref[:, D_ATOM:]
    d2 = jnp.sum(vec * vec, axis=1, keepdims=True)
    dist = jnp.sqrt(d2 + 1e-12) + 1e-6
    n = lax.broadcasted_iota(jnp.int32, (1, NUM_RADIAL), 1).astype(jnp.float32) + 1.0
    rbf = jnp.sqrt(2.0 / CUTOFF) * jnp.sin(n * (jnp.pi / CUTOFF) * dist) / dist
    u = jnp.clip(dist / CUTOFF, 0.0, 1.0)
    u5 = u * u * u * u * u
    env = 1.0 + (-21.0) * u5 + 35.0 * (u5 * u) + (-15.0) * (u5 * u * u)
    env = jnp.where(u < 1.0, env, 0.0)
    rbf = rbf * env
    re = jnp.dot(rbf.astype(jnp.bfloat16), wr_ref[...],
                 preferred_element_type=jnp.float32)
    z = (jnp.dot(hs_ref[:, :D_ATOM].astype(jnp.bfloat16), ws_ref[...],
                 preferred_element_type=jnp.float32)
         + jnp.dot(hd_ref[:, :D_ATOM].astype(jnp.bfloat16), wd_ref[...],
                   preferred_element_type=jnp.float32)
         + jnp.dot(re.astype(jnp.bfloat16), we_ref[...],
                   preferred_element_type=jnp.float32))
    m_ref[...] = (z * jax.nn.sigmoid(z)).astype(jnp.bfloat16)
    re_ref[...] = re


def _edge_init(hs, hd, W_rbf, We_s, We_d, We_r):
    grid = (EP // TE,)
    return pl.pallas_call(
        _edge_init_body,
        grid=grid,
        in_specs=[
            pl.BlockSpec((TE, D_ATOM + 128), lambda i: (i, 0)),
            pl.BlockSpec((TE, D_ATOM + 128), lambda i: (i, 0)),
            pl.BlockSpec((NUM_RADIAL, EMB_RBF), lambda i: (0, 0)),
            pl.BlockSpec((D_ATOM, D_EDGE), lambda i: (0, 0)),
            pl.BlockSpec((D_ATOM, D_EDGE), lambda i: (0, 0)),
            pl.BlockSpec((EMB_RBF, D_EDGE), lambda i: (0, 0)),
        ],
        out_specs=[
            pl.BlockSpec((TE, D_EDGE), lambda i: (i, 0)),
            pl.BlockSpec((TE, EMB_RBF), lambda i: (i, 0)),
        ],
        out_shape=[
            jax.ShapeDtypeStruct((EP, D_EDGE), jnp.bfloat16),
            jax.ShapeDtypeStruct((EP, EMB_RBF), jnp.float32),
        ],
    )(hs, hd, W_rbf, We_s, We_d, We_r)


def _edge_mix_body(m_ref, re_ref, wg_ref, w1_ref, w2_ref, wa_ref,
                   mmid_ref, amsg_ref):
    m0 = m_ref[...]
    gate = jnp.dot(re_ref[...].astype(jnp.bfloat16), wg_ref[...],
                   preferred_element_type=jnp.float32)
    z1 = jnp.dot(m0, w1_ref[...], preferred_element_type=jnp.float32)
    m2 = z1 * jax.nn.sigmoid(z1) * gate
    z2 = jnp.dot(m2.astype(jnp.bfloat16), w2_ref[...],
                 preferred_element_type=jnp.float32)
    m2 = z2 * jax.nn.sigmoid(z2)
    mm = (m0.astype(jnp.float32) + m2) * INV_SQRT2
    mmb = mm.astype(jnp.bfloat16)
    mmid_ref[...] = mmb
    amsg_ref[...] = jnp.dot(mmb, wa_ref[...],
                            preferred_element_type=jnp.float32).astype(jnp.bfloat16)


def _edge_mix(m, rbf_emb, Wgate, Wm1, Wm2, Wam):
    grid = (EP // TE,)
    return pl.pallas_call(
        _edge_mix_body,
        grid=grid,
        in_specs=[
            pl.BlockSpec((TE, D_EDGE), lambda i: (i, 0)),
            pl.BlockSpec((TE, EMB_RBF), lambda i: (i, 0)),
            pl.BlockSpec((EMB_RBF, D_EDGE), lambda i: (0, 0)),
            pl.BlockSpec((D_EDGE, D_EDGE), lambda i: (0, 0)),
            pl.BlockSpec((D_EDGE, D_EDGE), lambda i: (0, 0)),
            pl.BlockSpec((D_EDGE, D_ATOM), lambda i: (0, 0)),
        ],
        out_specs=[
            pl.BlockSpec((TE, D_EDGE), lambda i: (i, 0)),
            pl.BlockSpec((TE, D_ATOM), lambda i: (i, 0)),
        ],
        out_shape=[
            jax.ShapeDtypeStruct((EP, D_EDGE), jnp.bfloat16),
            jax.ShapeDtypeStruct((EP, D_ATOM), jnp.bfloat16),
        ],
    )(m, rbf_emb, Wgate, Wm1, Wm2, Wam)


def _atom_body(starts_ref, amsg_hbm, dst_hbm, h_ref, wh_ref, hnew_ref,
               hdst_hbm, am0, am1, ds0, ds1, agg_ref, hd0, hd1,
               sema0, sema1, semd0, semd1, semr, semw0, semw1, expand=True):
    t = pl.program_id(0)
    start = starts_ref[t]
    end = starts_ref[t + 1]
    end = jnp.where(t == NT - 1, EP, end)
    c0 = start // CE
    nch = jnp.maximum(0, (end + CE - 1) // CE - c0)
    agg_ref[...] = jnp.zeros((TA, D_ATOM), jnp.float32)

    def issue(j, am, ds, sa, sd):
        off = pl.multiple_of((c0 + j) * CE, CE)
        pltpu.make_async_copy(amsg_hbm.at[pl.ds(off, CE)], am, sa).start()
        pltpu.make_async_copy(dst_hbm.at[pl.ds(off, CE)], ds, sd).start()

    def wait_in(am, ds, sa, sd):
        pltpu.make_async_copy(amsg_hbm.at[pl.ds(0, CE)], am, sa).wait()
        pltpu.make_async_copy(dst_hbm.at[pl.ds(0, CE)], ds, sd).wait()

    def process(am, ds):
        dstl = ds[...] - t * TA
        ids = lax.broadcasted_iota(jnp.int32, (TA, CE), 0)
        S = (ids == dstl[None, :]).astype(jnp.bfloat16)
        agg_ref[...] += jnp.dot(S, am[...], preferred_element_type=jnp.float32)

    @pl.when(nch > 0)
    def _():
        issue(0, am0, ds0, sema0, semd0)

    def pair(k, carry):
        j1 = 2 * k + 1

        @pl.when(j1 < nch)
        def _():
            issue(j1, am1, ds1, sema1, semd1)

        wait_in(am0, ds0, sema0, semd0)
        process(am0, ds0)

        @pl.when(j1 < nch)
        def _():
            @pl.when(j1 + 1 < nch)
            def _():
                issue(j1 + 1, am0, ds0, sema0, semd0)

            wait_in(am1, ds1, sema1, semd1)
            process(am1, ds1)

        return carry

    lax.fori_loop(0, (nch + 1) // 2, pair, 0)
    z = jnp.dot(agg_ref[...], wh_ref[...], preferred_element_type=jnp.float32)
    hnew = h_ref[...] + z * jax.nn.sigmoid(z)
    hnew_ref[...] = hnew
    if not expand:
        return
    hb = hnew.astype(jnp.bfloat16)

    def issue_d(j, ds, sd):
        off = pl.multiple_of((c0 + j) * CE, CE)
        pltpu.make_async_copy(dst_hbm.at[pl.ds(off, CE)], ds, sd).start()

    def wait_d(ds, sd):
        pltpu.make_async_copy(dst_hbm.at[pl.ds(0, CE)], ds, sd).wait()

    def expand_write(j, ds, hdbuf, semw):
        off = pl.multiple_of((c0 + j) * CE, CE)
        dstl = ds[...] - t * TA
        ids2 = lax.broadcasted_iota(jnp.int32, (CE, TA), 1)
        G = (ids2 == dstl[:, None]).astype(jnp.bfloat16)
        hd = jnp.dot(G, hb, preferred_element_type=jnp.float32).astype(jnp.bfloat16)

        @pl.when(j >= 2)
        def _():
            pltpu.make_async_copy(hdbuf, hdst_hbm.at[pl.ds(0, CE)], semw).wait()

        shared = jnp.logical_and(j == 0, start % CE != 0)

        @pl.when(shared)
        def _():
            cr = pltpu.make_async_copy(hdst_hbm.at[pl.ds(off, CE)], hdbuf, semr)
            cr.start()
            cr.wait()
            hdbuf[...] = hdbuf[...] + hd

        @pl.when(jnp.logical_not(shared))
        def _():
            hdbuf[...] = hd

        pltpu.make_async_copy(hdbuf, hdst_hbm.at[pl.ds(off, CE)], semw).start()

    @pl.when(nch > 0)
    def _():
        issue_d(0, ds0, semd0)

    def pair2(k, carry):
        j1 = 2 * k + 1

        @pl.when(j1 < nch)
        def _():
            issue_d(j1, ds1, semd1)

        wait_d(ds0, semd0)
        expand_write(2 * k, ds0, hd0, semw0)

        @pl.when(j1 < nch)
        def _():
            @pl.when(j1 + 1 < nch)
            def _():
                issue_d(j1 + 1, ds0, semd0)

            wait_d(ds1, semd1)
            expand_write(j1, ds1, hd1, semw1)

        return carry

    lax.fori_loop(0, (nch + 1) // 2, pair2, 0)

    @pl.when(nch >= 1)
    def _():
        pltpu.make_async_copy(hd0, hdst_hbm.at[pl.ds(0, CE)], semw0).wait()

    @pl.when(nch >= 2)
    def _():
        pltpu.make_async_copy(hd1, hdst_hbm.at[pl.ds(0, CE)], semw1).wait()


def _atom_update(starts, amsg, dst_m, h, Wh, expand=True):
    grid_spec = pltpu.PrefetchScalarGridSpec(
        num_scalar_prefetch=1,
        grid=(NT,),
        in_specs=[
            pl.BlockSpec(memory_space=pl.ANY),
            pl.BlockSpec(memory_space=pl.ANY),
            pl.BlockSpec((TA, D_ATOM), lambda t, starts: (t, 0)),
            pl.BlockSpec((D_ATOM, D_ATOM), lambda t, starts: (0, 0)),
        ],
        out_specs=[
            pl.BlockSpec((TA, D_ATOM), lambda t, starts: (t, 0)),
            pl.BlockSpec(memory_space=pl.ANY),
        ],
        scratch_shapes=[
            pltpu.VMEM((CE, D_ATOM), jnp.bfloat16),
            pltpu.VMEM((CE, D_ATOM), jnp.bfloat16),
            pltpu.VMEM((CE,), jnp.int32),
            pltpu.VMEM((CE,), jnp.int32),
            pltpu.VMEM((TA, D_ATOM), jnp.float32),
            pltpu.VMEM((CE, D_ATOM), jnp.bfloat16),
            pltpu.VMEM((CE, D_ATOM), jnp.bfloat16),
            pltpu.SemaphoreType.DMA,
            pltpu.SemaphoreType.DMA,
            pltpu.SemaphoreType.DMA,
            pltpu.SemaphoreType.DMA,
            pltpu.SemaphoreType.DMA,
            pltpu.SemaphoreType.DMA,
            pltpu.SemaphoreType.DMA,
        ],
    )
    return pl.pallas_call(
        functools.partial(_atom_body, expand=expand),
        grid_spec=grid_spec,
        out_shape=[
            jax.ShapeDtypeStruct((NPAD, D_ATOM), jnp.float32),
            jax.ShapeDtypeStruct((EP, D_ATOM), jnp.bfloat16),
        ],
    )(starts, amsg, dst_m, h, Wh)


def _edge_fused_body(hs_ref, hd_ref, m_ref, re_ref, ws_ref, wd_ref, wm_ref,
                     wg_ref, w1_ref, w2_ref, wa_ref, mmid_ref, amsg_ref):
    z = (jnp.dot(hs_ref[...].astype(jnp.bfloat16), ws_ref[...],
                 preferred_element_type=jnp.float32)
         + jnp.dot(hd_ref[...], wd_ref[...], preferred_element_type=jnp.float32)
         + jnp.dot(m_ref[...], wm_ref[...], preferred_element_type=jnp.float32))
    e = z * jax.nn.sigmoid(z)
    m0 = (m_ref[...].astype(jnp.float32) + e) * INV_SQRT2
    m0b = m0.astype(jnp.bfloat16)
    gate = jnp.dot(re_ref[...].astype(jnp.bfloat16), wg_ref[...],
                   preferred_element_type=jnp.float32)
    z1 = jnp.dot(m0b, w1_ref[...], preferred_element_type=jnp.float32)
    m2 = z1 * jax.nn.sigmoid(z1) * gate
    z2 = jnp.dot(m2.astype(jnp.bfloat16), w2_ref[...],
                 preferred_element_type=jnp.float32)
    m2 = z2 * jax.nn.sigmoid(z2)
    mm = (m0 + m2) * INV_SQRT2
    mmb = mm.astype(jnp.bfloat16)
    mmid_ref[...] = mmb
    amsg_ref[...] = jnp.dot(mmb, wa_ref[...],
                            preferred_element_type=jnp.float32).astype(jnp.bfloat16)


def _edge_fused(hs, hd, mmid, rbf_emb, We_s, We_d, We_m, Wgate, Wm1, Wm2, Wam):
    grid = (EP // TE,)
    return pl.pallas_call(
        _edge_fused_body,
        grid=grid,
        in_specs=[
            pl.BlockSpec((TE, D_ATOM), lambda i: (i, 0)),
            pl.BlockSpec((TE, D_ATOM), lambda i: (i, 0)),
            pl.BlockSpec((TE, D_EDGE), lambda i: (i, 0)),
            pl.BlockSpec((TE, EMB_RBF), lambda i: (i, 0)),
            pl.BlockSpec((D_ATOM, D_EDGE), lambda i: (0, 0)),
            pl.BlockSpec((D_ATOM, D_EDGE), lambda i: (0, 0)),
            pl.BlockSpec((D_EDGE, D_EDGE), lambda i: (0, 0)),
            pl.BlockSpec((EMB_RBF, D_EDGE), lambda i: (0, 0)),
            pl.BlockSpec((D_EDGE, D_EDGE), lambda i: (0, 0)),
            pl.BlockSpec((D_EDGE, D_EDGE), lambda i: (0, 0)),
            pl.BlockSpec((D_EDGE, D_ATOM), lambda i: (0, 0)),
        ],
        out_specs=[
            pl.BlockSpec((TE, D_EDGE), lambda i: (i, 0)),
            pl.BlockSpec((TE, D_ATOM), lambda i: (i, 0)),
        ],
        out_shape=[
            jax.ShapeDtypeStruct((EP, D_EDGE), jnp.bfloat16),
            jax.ShapeDtypeStruct((EP, D_ATOM), jnp.bfloat16),
        ],
    )(hs, hd, mmid, rbf_emb, We_s, We_d, We_m, Wgate, Wm1, Wm2, Wam)


def _edge_up_body(hs_ref, hd_ref, m_ref, ws_ref, wd_ref, wm_ref, out_ref):
    z = (jnp.dot(hs_ref[...].astype(jnp.bfloat16), ws_ref[...],
                 preferred_element_type=jnp.float32)
         + jnp.dot(hd_ref[...], wd_ref[...], preferred_element_type=jnp.float32)
         + jnp.dot(m_ref[...], wm_ref[...], preferred_element_type=jnp.float32))
    e = z * jax.nn.sigmoid(z)
    out_ref[...] = ((m_ref[...].astype(jnp.float32) + e)
                    * INV_SQRT2).astype(jnp.bfloat16)


def _edge_up(hs, hd, mmid, We_s, We_d, We_m):
    grid = (EP // TE,)
    return pl.pallas_call(
        _edge_up_body,
        grid=grid,
        in_specs=[
            pl.BlockSpec((TE, D_ATOM), lambda i: (i, 0)),
            pl.BlockSpec((TE, D_ATOM), lambda i: (i, 0)),
            pl.BlockSpec((TE, D_EDGE), lambda i: (i, 0)),
            pl.BlockSpec((D_ATOM, D_EDGE), lambda i: (0, 0)),
            pl.BlockSpec((D_ATOM, D_EDGE), lambda i: (0, 0)),
            pl.BlockSpec((D_EDGE, D_EDGE), lambda i: (0, 0)),
        ],
        out_specs=pl.BlockSpec((TE, D_EDGE), lambda i: (i, 0)),
        out_shape=jax.ShapeDtypeStruct((EP, D_EDGE), jnp.bfloat16),
    )(hs, hd, mmid, We_s, We_d, We_m)


def _energy_body(h_ref, b_ref, w_ref, out_ref):
    t = pl.program_id(0)

    @pl.when(t == 0)
    def _():
        out_ref[...] = jnp.zeros_like(out_ref)

    e = jnp.dot(h_ref[...], w_ref[...], preferred_element_type=jnp.float32)
    bids = b_ref[0, 0, :]
    gids = lax.broadcasted_iota(jnp.int32, (NG, TA), 0)
    S = (gids == bids[None, :]).astype(jnp.float32)
    out_ref[...] += jnp.dot(S, e, preferred_element_type=jnp.float32)


def _energy(h, bids3, Wout_pad):
    return pl.pallas_call(
        _energy_body,
        grid=(NT,),
        in_specs=[
            pl.BlockSpec((TA, D_ATOM), lambda t: (t, 0)),
            pl.BlockSpec((1, 1, TA), lambda t: (t, 0, 0)),
            pl.BlockSpec((D_ATOM, 128), lambda t: (0, 0)),
        ],
        out_specs=pl.BlockSpec((NG, 128), lambda t: (0, 0)),
        out_shape=jax.ShapeDtypeStruct((NG, 128), jnp.float32),
    )(h, bids3, Wout_pad)


def kernel(atomic_numbers, pos, edge_index, batch_ids, emb_table, W_rbf,
           W_edge, W_m1, W_m2, W_gate, W_am, W_h, W_e, W_out):
    src = edge_index[0].astype(jnp.int32)
    dst = edge_index[1].astype(jnp.int32)
    perm = jnp.argsort(dst)
    dsts = dst[perm]
    srcs = src[perm]
    pad_e = EP - E
    zpad = jnp.zeros((pad_e,), jnp.int32)
    src_g = jnp.concatenate([srcs, zpad])
    dst_g = jnp.concatenate([dsts, zpad])
    dst_m = jnp.concatenate([dsts, jnp.full((pad_e,), 1 << 20, jnp.int32)])
    starts = jnp.searchsorted(
        dsts, jnp.arange(NT + 1, dtype=jnp.int32) * TA).astype(jnp.int32)

    an_pad = jnp.concatenate(
        [atomic_numbers.astype(jnp.int32), jnp.zeros((NPAD - N,), jnp.int32)])
    pos_pad = jnp.zeros((NPAD, 128), jnp.float32).at[:N, :3].set(pos)

    bf = jnp.bfloat16
    h = _sc_gather(emb_table, an_pad, 160)
    hp = jnp.concatenate([h, pos_pad], axis=1)
    hps = _sc_gather(hp, src_g, 160)
    hpd = _sc_gather(hp, dst_g, 160)

    m, rbf_emb = _edge_init(hps, hpd, W_rbf.astype(bf),
                            W_edge[:D_ATOM].astype(bf),
                            W_edge[D_ATOM:2 * D_ATOM].astype(bf),
                            W_edge[2 * D_ATOM:].astype(bf))

    mmid, amsg = _edge_mix(m, rbf_emb, W_gate[0].astype(bf),
                           W_m1[0].astype(bf), W_m2[0].astype(bf),
                           W_am[0].astype(bf))
    for i in range(N_BLOCKS):
        last = i == N_BLOCKS - 1
        h, hd = _atom_update(starts, amsg, dst_m, h, W_h[i], expand=not last)
        if not last:
            hs = _sc_gather(h, src_g, 160)
            mmid, amsg = _edge_fused(
                hs, hd, mmid, rbf_emb,
                W_e[i][:D_ATOM].astype(bf),
                W_e[i][D_ATOM:2 * D_ATOM].astype(bf),
                W_e[i][2 * D_ATOM:].astype(bf),
                W_gate[i + 1].astype(bf), W_m1[i + 1].astype(bf),
                W_m2[i + 1].astype(bf), W_am[i + 1].astype(bf))

    bids3 = jnp.concatenate(
        [batch_ids.astype(jnp.int32),
         jnp.full((NPAD - N,), NG, jnp.int32)]).reshape(NT, 1, TA)
    wout_pad = jnp.zeros((D_ATOM, 128), jnp.float32).at[:, :1].set(W_out)
    energy = _energy(h, bids3, wout_pad)
    return energy[:, 0]
